# baseline (speedup 1.0000x reference)
import functools

import jax
import jax.numpy as jnp
from jax import lax
from jax.experimental import pallas as pl
from jax.experimental.pallas import tpu as pltpu
from jax.experimental.pallas import tpu_sc as plsc

B, L, D = 16384, 7, 64
DP = 2 * D
V = 1000000
R = B * L
NW = 32
JPW = R // NW
CHUNK = 256
NCHUNK = JPW // CHUNK
GSZ = 128
NSUB = CHUNK // GSZ
NLANE = 16
NVPR = D // NLANE


VB = 2048


def _tc_transpose_body(tt_ref, out_ref):
    eye = jax.lax.broadcasted_iota(jnp.int32, (D, D), 0)
    eye = (eye == jax.lax.broadcasted_iota(jnp.int32, (D, D), 1)).astype(
        jnp.float32)
    t = jax.lax.dot_general(tt_ref[...], eye, (((0,), (0,)), ((), ())),
                            preferred_element_type=jnp.float32)
    out_ref[...] = jnp.concatenate([t, t], axis=1)


def _tc_transpose(tt):
    return pl.pallas_call(
        _tc_transpose_body,
        grid=((V + VB - 1) // VB,),
        in_specs=[pl.BlockSpec((D, VB), lambda i: (0, i))],
        out_specs=pl.BlockSpec((VB, DP), lambda i: (i, 0)),
        out_shape=jax.ShapeDtypeStruct((V, DP), jnp.float32),
    )(tt)


def _build_sc_kernel():
    mesh = plsc.VectorSubcoreMesh(core_axis_name="c", subcore_axis_name="s")

    @functools.partial(
        pl.kernel,
        mesh=mesh,
        out_type=jax.ShapeDtypeStruct((L, B, D), jnp.float32),
        scratch_types=[
            pltpu.VMEM((L, CHUNK), jnp.int32),
            pltpu.VMEM((CHUNK, DP), jnp.float32),
            pltpu.VMEM((CHUNK, D), jnp.float32),
            pltpu.VMEM((L, D), jnp.float32),
            pltpu.SemaphoreType.DMA,
        ],
    )
    def sc_embed(xt_hbm, tok_hbm, pos_hbm, out_hbm,
                 idx_v, rows_v, out_v, pos_v, sem):
        wid = lax.axis_index("s") * 2 + lax.axis_index("c")
        pltpu.sync_copy(pos_hbm, pos_v)

        for kck in range(NCHUNK):
            m = wid * NCHUNK + kck
            l = m // (B // CHUNK)
            b0 = (m % (B // CHUNK)) * CHUNK
            pltpu.sync_copy(xt_hbm.at[:, pl.ds(b0, CHUNK)], idx_v)
            copies = []
            for j in range(NSUB):
                copies.append(pltpu.async_copy(
                    tok_hbm.at[idx_v.at[l, pl.ds(j * GSZ, GSZ)]],
                    rows_v.at[pl.ds(j * GSZ, GSZ)],
                    sem))
            for cp in copies:
                cp.wait()

            pv = [pos_v[l, pl.ds(c * NLANE, NLANE)] for c in range(NVPR)]

            def body(g, carry):
                for u in range(4):
                    r = g * 4 + u
                    for c in range(NVPR):
                        sl = pl.ds(c * NLANE, NLANE)
                        out_v[r, sl] = rows_v[r, sl] + pv[c]
                return carry

            lax.fori_loop(0, CHUNK // 4, body, 0)
            pltpu.sync_copy(out_v, out_hbm.at[l].at[pl.ds(b0, CHUNK)])

    return sc_embed


_sc_embed = _build_sc_kernel()


def kernel(x, token_table, pos_table):
    xt = x.astype(jnp.int32).T
    tok_pad = _tc_transpose(token_table.T)
    out = _sc_embed(xt, tok_pad, pos_table)
    return out.transpose(1, 0, 2)


# speedup vs baseline: 1.4500x; 1.4500x over previous
import functools

import jax
import jax.numpy as jnp
from jax import lax
from jax.experimental import pallas as pl
from jax.experimental.pallas import tpu as pltpu
from jax.experimental.pallas import tpu_sc as plsc

B, L, D = 16384, 7, 64
DP = 2 * D
V = 1000000
R = B * L
NW = 32
JPW = R // NW
CHUNK = 256
NCHUNK = JPW // CHUNK
GSZ = 128
NSUB = CHUNK // GSZ
NLANE = 16
NVPR = D // NLANE


VB = 8192


def _tc_transpose_body(tt_ref, out_ref):
    eye = jax.lax.broadcasted_iota(jnp.int32, (D, D), 0)
    eye = (eye == jax.lax.broadcasted_iota(jnp.int32, (D, D), 1)).astype(
        jnp.float32)
    t = jax.lax.dot_general(tt_ref[...], eye, (((0,), (0,)), ((), ())),
                            preferred_element_type=jnp.float32)
    out_ref[...] = jnp.concatenate([t, t], axis=1)


def _tc_transpose(tt):
    return pl.pallas_call(
        _tc_transpose_body,
        grid=((V + VB - 1) // VB,),
        in_specs=[pl.BlockSpec((D, VB), lambda i: (0, i))],
        out_specs=pl.BlockSpec((VB, DP), lambda i: (i, 0)),
        out_shape=jax.ShapeDtypeStruct((V, DP), jnp.float32),
    )(tt)


def _build_sc_kernel():
    mesh = plsc.VectorSubcoreMesh(core_axis_name="c", subcore_axis_name="s")

    @functools.partial(
        pl.kernel,
        mesh=mesh,
        out_type=jax.ShapeDtypeStruct((L, B, D), jnp.float32),
        scratch_types=[
            pltpu.VMEM((L, CHUNK), jnp.int32),
            pltpu.VMEM((CHUNK, DP), jnp.float32),
            pltpu.VMEM((CHUNK, D), jnp.float32),
            pltpu.VMEM((L, D), jnp.float32),
            pltpu.SemaphoreType.DMA,
        ],
    )
    def sc_embed(xt_hbm, tok_hbm, pos_hbm, out_hbm,
                 idx_v, rows_v, out_v, pos_v, sem):
        wid = lax.axis_index("s") * 2 + lax.axis_index("c")
        pltpu.sync_copy(pos_hbm, pos_v)

        for kck in range(NCHUNK):
            m = wid * NCHUNK + kck
            l = m // (B // CHUNK)
            b0 = (m % (B // CHUNK)) * CHUNK
            pltpu.sync_copy(xt_hbm.at[:, pl.ds(b0, CHUNK)], idx_v)
            copies = []
            for j in range(NSUB):
                copies.append(pltpu.async_copy(
                    tok_hbm.at[idx_v.at[l, pl.ds(j * GSZ, GSZ)]],
                    rows_v.at[pl.ds(j * GSZ, GSZ)],
                    sem))
            for cp in copies:
                cp.wait()

            pv = [pos_v[l, pl.ds(c * NLANE, NLANE)] for c in range(NVPR)]

            def body(g, carry):
                for u in range(4):
                    r = g * 4 + u
                    for c in range(NVPR):
                        sl = pl.ds(c * NLANE, NLANE)
                        out_v[r, sl] = rows_v[r, sl] + pv[c]
                return carry

            lax.fori_loop(0, CHUNK // 4, body, 0)
            pltpu.sync_copy(out_v, out_hbm.at[l].at[pl.ds(b0, CHUNK)])

    return sc_embed


_sc_embed = _build_sc_kernel()


def kernel(x, token_table, pos_table):
    xt = x.astype(jnp.int32).T
    tok_pad = _tc_transpose(token_table.T)
    out = _sc_embed(xt, tok_pad, pos_table)
    return out.transpose(1, 0, 2)


# speedup vs baseline: 1.5754x; 1.0865x over previous
import functools

import jax
import jax.numpy as jnp
from jax import lax
from jax.experimental import pallas as pl
from jax.experimental.pallas import tpu as pltpu
from jax.experimental.pallas import tpu_sc as plsc

B, L, D = 16384, 7, 64
DP = 2 * D
V = 1000000
R = B * L
NW = 32
JPW = R // NW
CHUNK = 256
NCHUNK = JPW // CHUNK
GSZ = 128
NSUB = CHUNK // GSZ
NLANE = 16
NVPR = D // NLANE


VB = 16384


def _tc_transpose_body(tt_ref, out_ref):
    eye = jax.lax.broadcasted_iota(jnp.int32, (D, D), 0)
    eye = (eye == jax.lax.broadcasted_iota(jnp.int32, (D, D), 1)).astype(
        jnp.float32)
    t = jax.lax.dot_general(tt_ref[...], eye, (((0,), (0,)), ((), ())),
                            preferred_element_type=jnp.float32)
    out_ref[...] = jnp.concatenate([t, t], axis=1)


def _tc_transpose(tt):
    return pl.pallas_call(
        _tc_transpose_body,
        grid=((V + VB - 1) // VB,),
        in_specs=[pl.BlockSpec((D, VB), lambda i: (0, i))],
        out_specs=pl.BlockSpec((VB, DP), lambda i: (i, 0)),
        out_shape=jax.ShapeDtypeStruct((V, DP), jnp.float32),
    )(tt)


def _build_sc_kernel():
    mesh = plsc.VectorSubcoreMesh(core_axis_name="c", subcore_axis_name="s")

    @functools.partial(
        pl.kernel,
        mesh=mesh,
        out_type=jax.ShapeDtypeStruct((L, B, D), jnp.float32),
        scratch_types=[
            pltpu.VMEM((L, CHUNK), jnp.int32),
            pltpu.VMEM((CHUNK, DP), jnp.float32),
            pltpu.VMEM((CHUNK, D), jnp.float32),
            pltpu.VMEM((L, D), jnp.float32),
            pltpu.SemaphoreType.DMA,
        ],
    )
    def sc_embed(xt_hbm, tok_hbm, pos_hbm, out_hbm,
                 idx_v, rows_v, out_v, pos_v, sem):
        wid = lax.axis_index("s") * 2 + lax.axis_index("c")
        pltpu.sync_copy(pos_hbm, pos_v)

        for kck in range(NCHUNK):
            m = wid * NCHUNK + kck
            l = m // (B // CHUNK)
            b0 = (m % (B // CHUNK)) * CHUNK
            pltpu.sync_copy(xt_hbm.at[:, pl.ds(b0, CHUNK)], idx_v)
            copies = []
            for j in range(NSUB):
                copies.append(pltpu.async_copy(
                    tok_hbm.at[idx_v.at[l, pl.ds(j * GSZ, GSZ)]],
                    rows_v.at[pl.ds(j * GSZ, GSZ)],
                    sem))
            for cp in copies:
                cp.wait()

            pv = [pos_v[l, pl.ds(c * NLANE, NLANE)] for c in range(NVPR)]

            def body(g, carry):
                for u in range(4):
                    r = g * 4 + u
                    for c in range(NVPR):
                        sl = pl.ds(c * NLANE, NLANE)
                        out_v[r, sl] = rows_v[r, sl] + pv[c]
                return carry

            lax.fori_loop(0, CHUNK // 4, body, 0)
            pltpu.sync_copy(out_v, out_hbm.at[l].at[pl.ds(b0, CHUNK)])

    return sc_embed


_sc_embed = _build_sc_kernel()


def kernel(x, token_table, pos_table):
    xt = x.astype(jnp.int32).T
    tok_pad = _tc_transpose(token_table.T)
    out = _sc_embed(xt, tok_pad, pos_table)
    return out.transpose(1, 0, 2)


# speedup vs baseline: 1.6504x; 1.0476x over previous
import functools

import jax
import jax.numpy as jnp
from jax import lax
from jax.experimental import pallas as pl
from jax.experimental.pallas import tpu as pltpu
from jax.experimental.pallas import tpu_sc as plsc

B, L, D = 16384, 7, 64
DP = 2 * D
V = 1000000
R = B * L
NW = 32
JPW = R // NW
CHUNK = 256
NCHUNK = JPW // CHUNK
GSZ = 128
NSUB = CHUNK // GSZ
NLANE = 16
NVPR = D // NLANE


VB = 16384


def _tc_transpose_body(tt_ref, out_ref):
    eye = jax.lax.broadcasted_iota(jnp.int32, (D, D), 0)
    eye = (eye == jax.lax.broadcasted_iota(jnp.int32, (D, D), 1)).astype(
        jnp.float32)
    t = jax.lax.dot_general(tt_ref[...], eye, (((0,), (0,)), ((), ())),
                            preferred_element_type=jnp.float32)
    out_ref[...] = jnp.concatenate([t, t], axis=1)


def _tc_transpose(tt):
    return pl.pallas_call(
        _tc_transpose_body,
        grid=((V + VB - 1) // VB,),
        in_specs=[pl.BlockSpec((D, VB), lambda i: (0, i))],
        out_specs=pl.BlockSpec((VB, DP), lambda i: (i, 0)),
        out_shape=jax.ShapeDtypeStruct((V, DP), jnp.float32),
    )(tt)


def _build_sc_kernel():
    mesh = plsc.VectorSubcoreMesh(core_axis_name="c", subcore_axis_name="s")

    @functools.partial(
        pl.kernel,
        mesh=mesh,
        out_type=jax.ShapeDtypeStruct((L, B, D), jnp.float32),
        scratch_types=[
            pltpu.VMEM((L, CHUNK), jnp.int32),
            pltpu.VMEM((L, CHUNK), jnp.int32),
            pltpu.VMEM((CHUNK, DP), jnp.float32),
            pltpu.VMEM((CHUNK, DP), jnp.float32),
            pltpu.VMEM((CHUNK, D), jnp.float32),
            pltpu.VMEM((L, D), jnp.float32),
            pltpu.SemaphoreType.DMA,
            pltpu.SemaphoreType.DMA,
        ],
    )
    def sc_embed(xt_hbm, tok_hbm, pos_hbm, out_hbm,
                 idx0, idx1, rows0, rows1, out_v, pos_v, sem0, sem1):
        wid = lax.axis_index("s") * 2 + lax.axis_index("c")
        idxs, rows, sems = (idx0, idx1), (rows0, rows1), (sem0, sem1)
        pltpu.sync_copy(pos_hbm, pos_v)

        def lb0(kck):
            m = wid * NCHUNK + kck
            return m // (B // CHUNK), (m % (B // CHUNK)) * CHUNK

        def stage(kck):
            l, b0 = lb0(kck)
            p = kck % 2
            pltpu.sync_copy(xt_hbm.at[:, pl.ds(b0, CHUNK)], idxs[p])
            return [pltpu.async_copy(
                tok_hbm.at[idxs[p].at[l, pl.ds(j * GSZ, GSZ)]],
                rows[p].at[pl.ds(j * GSZ, GSZ)],
                sems[p]) for j in range(NSUB)]

        inflight = {0: stage(0)}
        for kck in range(NCHUNK):
            if kck + 1 < NCHUNK:
                inflight[kck + 1] = stage(kck + 1)
            for cp in inflight.pop(kck):
                cp.wait()
            l, b0 = lb0(kck)
            rows_v = rows[kck % 2]
            pv = [pos_v[l, pl.ds(c * NLANE, NLANE)] for c in range(NVPR)]

            def body(g, carry):
                for u in range(4):
                    r = g * 4 + u
                    for c in range(NVPR):
                        sl = pl.ds(c * NLANE, NLANE)
                        out_v[r, sl] = rows_v[r, sl] + pv[c]
                return carry

            lax.fori_loop(0, CHUNK // 4, body, 0)
            pltpu.sync_copy(out_v, out_hbm.at[l].at[pl.ds(b0, CHUNK)])

    return sc_embed


_sc_embed = _build_sc_kernel()


def kernel(x, token_table, pos_table):
    xt = x.astype(jnp.int32).T
    tok_pad = _tc_transpose(token_table.T)
    out = _sc_embed(xt, tok_pad, pos_table)
    return out.transpose(1, 0, 2)
